# baseline (speedup 1.0000x reference)
import jax
import jax.numpy as jnp
from jax import lax
from jax.experimental import pallas as pl
from jax.experimental.pallas import tpu as pltpu

B = 16384
D_FACE = 64
D_BODY = 74
D_ID = D_BODY - 10
UNROLL = 16


def _gather_kernel(idx_f, idx_b, face_hbm, body_hbm,
                   out_f, out_b, out_s, rows_b, sem_f, sem_b):
    def issue(j0, _):
        for u in range(UNROLL):
            j = j0 * UNROLL + u
            pltpu.make_async_copy(
                face_hbm.at[pl.ds(idx_f[j], 1)],
                out_f.at[pl.ds(j, 1)], sem_f).start()
            pltpu.make_async_copy(
                body_hbm.at[pl.ds(idx_b[j], 1)],
                rows_b.at[pl.ds(j, 1)], sem_b).start()
        return ()
    lax.fori_loop(0, B // UNROLL, issue, ())

    pltpu.make_async_copy(face_hbm.at[pl.ds(0, B)], out_f, sem_f).wait()
    pltpu.make_async_copy(body_hbm.at[pl.ds(0, B)], rows_b, sem_b).wait()

    out_b[...] = rows_b[:, :D_ID]
    out_s[...] = rows_b[:, D_ID:D_BODY]


def kernel(rand_id_face, rand_id_body, id_face_sampler, id_shape_sampler_body):
    grid_spec = pltpu.PrefetchScalarGridSpec(
        num_scalar_prefetch=2,
        in_specs=[
            pl.BlockSpec(memory_space=pltpu.MemorySpace.HBM),
            pl.BlockSpec(memory_space=pltpu.MemorySpace.HBM),
        ],
        out_specs=[
            pl.BlockSpec(memory_space=pltpu.MemorySpace.HBM),
            pl.BlockSpec(memory_space=pltpu.MemorySpace.VMEM),
            pl.BlockSpec(memory_space=pltpu.MemorySpace.VMEM),
        ],
        scratch_shapes=[
            pltpu.VMEM((B, D_BODY), jnp.float32),
            pltpu.SemaphoreType.DMA,
            pltpu.SemaphoreType.DMA,
        ],
    )
    f = pl.pallas_call(
        _gather_kernel,
        grid_spec=grid_spec,
        out_shape=(
            jax.ShapeDtypeStruct((B, D_FACE), jnp.float32),
            jax.ShapeDtypeStruct((B, D_ID), jnp.float32),
            jax.ShapeDtypeStruct((B, 10), jnp.float32),
        ),
    )
    return f(rand_id_face, rand_id_body, id_face_sampler, id_shape_sampler_body)


# speedup vs baseline: 1.1406x; 1.1406x over previous
import jax
import jax.numpy as jnp
from jax import lax
from jax.experimental import pallas as pl
from jax.experimental.pallas import tpu as pltpu

B = 16384
D_FACE = 64
D_BODY = 74
D_ID = D_BODY - 10
UNROLL = 16


def _gather_kernel(idx_f, idx_b, face_hbm, body_hbm,
                   out_f, out_b, out_s, rows_b, sem_f, sem_b):
    def issue(j0, _):
        for u in range(UNROLL):
            j = j0 * UNROLL + u
            pltpu.make_async_copy(
                face_hbm.at[pl.ds(idx_f[j], 1)],
                out_f.at[pl.ds(j, 1)], sem_f).start()
            pltpu.make_async_copy(
                body_hbm.at[pl.ds(idx_b[j], 1)],
                rows_b.at[pl.ds(j, 1)], sem_b).start()
        return ()
    lax.fori_loop(0, B // UNROLL, issue, ())

    pltpu.make_async_copy(face_hbm.at[pl.ds(0, B)], out_f, sem_f).wait()
    pltpu.make_async_copy(body_hbm.at[pl.ds(0, B)], rows_b, sem_b).wait()

    out_b[...] = rows_b[:, :D_ID]
    out_s[...] = rows_b[:, D_ID:D_BODY]


def kernel(rand_id_face, rand_id_body, id_face_sampler, id_shape_sampler_body):
    grid_spec = pltpu.PrefetchScalarGridSpec(
        num_scalar_prefetch=2,
        in_specs=[
            pl.BlockSpec(memory_space=pltpu.MemorySpace.HBM),
            pl.BlockSpec(memory_space=pltpu.MemorySpace.HBM),
        ],
        scratch_shapes=[
            pltpu.VMEM((B, D_BODY), jnp.float32),
            pltpu.SemaphoreType.DMA,
            pltpu.SemaphoreType.DMA,
        ],
    )
    f = pl.pallas_call(
        _gather_kernel,
        grid_spec=grid_spec,
        out_shape=(
            jax.ShapeDtypeStruct((B, D_FACE), jnp.float32),
            jax.ShapeDtypeStruct((B, D_ID), jnp.float32),
            jax.ShapeDtypeStruct((B, 10), jnp.float32),
        ),
    )
    return f(rand_id_face, rand_id_body, id_face_sampler, id_shape_sampler_body)
